# Initial kernel scaffold; baseline (speedup 1.0000x reference)
import jax
import jax.numpy as jnp
from jax.experimental import pallas as pl


def kernel(x):
    raise NotImplementedError("write your pallas kernel here")


# speedup vs baseline: 1.0331x; 1.0331x over previous
import jax
import jax.numpy as jnp
from jax.experimental import pallas as pl


_ROWS = 8192
_COLS = 1024
_BLOCK_ROWS = 1024


def _copy_kernel(x_ref, o_ref):
    o_ref[...] = x_ref[...]


def kernel(x):
    n_blocks = _ROWS // _BLOCK_ROWS
    gathered = pl.pallas_call(
        _copy_kernel,
        grid=(n_blocks,),
        in_specs=[pl.BlockSpec((_BLOCK_ROWS, _COLS), lambda i: (i, 0))],
        out_specs=pl.BlockSpec((_BLOCK_ROWS, _COLS), lambda i: (i, 0)),
        out_shape=jax.ShapeDtypeStruct((_ROWS, _COLS), x.dtype),
    )(x)
    sizes = jnp.array([_ROWS], dtype=jnp.int32)
    return (gathered, sizes)
